# baseline (speedup 1.0000x reference)
import functools

import jax
import jax.numpy as jnp
from jax import lax
from jax.experimental import pallas as pl
from jax.experimental.pallas import tpu as pltpu
from jax.experimental.pallas import tpu_sc as plsc

N = 10000
E = 320000
F = 128
D1 = 160
NC, NS = 2, 16
CH = 32
CPT = 320
NBUF = 4
EPAD = NC * NS * CPT * CH
ACC_ROWS = 10016
ZR = 626
RB = 5000

_P = jax.lax.Precision.HIGHEST


def _tc1_body(x_ref, wl_ref, e_ref, wr_ref, p_ref, r_ref):
    x = x_ref[...]
    p = lax.dot(x.astype(jnp.bfloat16), wl_ref[...].astype(jnp.bfloat16),
                preferred_element_type=jnp.float32) + e_ref[...]
    p_ref[...] = p.astype(jnp.bfloat16)
    r_ref[...] = lax.dot(x, wr_ref[...], precision=_P)


def _tc1(x, w1aug, e144, w1r):
    return pl.pallas_call(
        _tc1_body,
        grid=(N // RB,),
        in_specs=[
            pl.BlockSpec((RB, F), lambda i: (i, 0)),
            pl.BlockSpec((F, D1), lambda i: (0, 0)),
            pl.BlockSpec((1, D1), lambda i: (0, 0)),
            pl.BlockSpec((F, F), lambda i: (0, 0)),
        ],
        out_specs=[
            pl.BlockSpec((RB, D1), lambda i: (i, 0)),
            pl.BlockSpec((RB, F), lambda i: (i, 0)),
        ],
        out_shape=[
            jax.ShapeDtypeStruct((N, D1), jnp.bfloat16),
            jax.ShapeDtypeStruct((N, F), jnp.float32),
        ],
    )(x, w1aug, e144, w1r)


def _tc2_body(s_ref, r_ref, wl_ref, wr_ref, b1_ref, p_ref, r2_ref, deg_ref):
    s = s_ref[0].astype(jnp.float32) + s_ref[1].astype(jnp.float32)
    deg = s[:, F:F + 1]
    mean = s[:, :F] / jnp.maximum(deg, 1.0)
    h = jnp.maximum(mean + r_ref[...] + b1_ref[...], 0.0)
    p_ref[...] = lax.dot(h.astype(jnp.bfloat16),
                         wl_ref[...].astype(jnp.bfloat16),
                         preferred_element_type=jnp.float32
                         ).astype(jnp.bfloat16)
    r2_ref[...] = lax.dot(h, wr_ref[...], precision=_P)
    deg_ref[...] = jnp.broadcast_to(deg, (RB, F))


def _tc2(s1, r1, w2l, w2r, b1):
    return pl.pallas_call(
        _tc2_body,
        grid=(N // RB,),
        in_specs=[
            pl.BlockSpec((2, RB, D1), lambda i: (0, i, 0)),
            pl.BlockSpec((RB, F), lambda i: (i, 0)),
            pl.BlockSpec((F, F), lambda i: (0, 0)),
            pl.BlockSpec((F, F), lambda i: (0, 0)),
            pl.BlockSpec((1, F), lambda i: (0, 0)),
        ],
        out_specs=[
            pl.BlockSpec((RB, F), lambda i: (i, 0)),
            pl.BlockSpec((RB, F), lambda i: (i, 0)),
            pl.BlockSpec((RB, F), lambda i: (i, 0)),
        ],
        out_shape=[
            jax.ShapeDtypeStruct((N, F), jnp.bfloat16),
            jax.ShapeDtypeStruct((N, F), jnp.float32),
            jax.ShapeDtypeStruct((N, F), jnp.float32),
        ],
    )(s1, r1, w2l, w2r, b1)


def _tc3_body(s_ref, r_ref, deg_ref, b2_ref, o_ref):
    s = s_ref[0].astype(jnp.float32) + s_ref[1].astype(jnp.float32)
    o = s / jnp.maximum(deg_ref[...], 1.0) + r_ref[...] + b2_ref[...]
    m = jnp.max(o, axis=1, keepdims=True)
    z = o - m
    o_ref[...] = z - jnp.log(jnp.sum(jnp.exp(z), axis=1, keepdims=True))


def _tc3(s2, r2, degb, b2):
    return pl.pallas_call(
        _tc3_body,
        grid=(N // RB,),
        in_specs=[
            pl.BlockSpec((2, RB, F), lambda i: (0, i, 0)),
            pl.BlockSpec((RB, F), lambda i: (i, 0)),
            pl.BlockSpec((RB, F), lambda i: (i, 0)),
            pl.BlockSpec((1, F), lambda i: (0, 0)),
        ],
        out_specs=pl.BlockSpec((RB, F), lambda i: (i, 0)),
        out_shape=jax.ShapeDtypeStruct((N, F), jnp.float32),
    )(s2, r2, degb, b2)


@functools.cache
def _make_segsum(d):
    mesh = plsc.VectorSubcoreMesh(core_axis_name="c", subcore_axis_name="s",
                                  num_cores=NC, num_subcores=NS)

    @functools.partial(
        pl.kernel,
        out_type=jax.ShapeDtypeStruct((NC, ACC_ROWS, d), jnp.bfloat16),
        mesh=mesh,
        scratch_types=[
            pltpu.VMEM((CPT, CH), jnp.int32),
            pltpu.VMEM((CPT, CH), jnp.int32),
            [pltpu.VMEM((CH, d), jnp.bfloat16) for _ in range(NBUF)],
            [pltpu.SemaphoreType.DMA for _ in range(NBUF)],
            [pltpu.SemaphoreType.DMA for _ in range(NBUF)],
            pltpu.VMEM_SHARED((ACC_ROWS, d), jnp.bfloat16),
            pltpu.VMEM_SHARED((N, d), jnp.bfloat16),
        ],
        compiler_params=pltpu.CompilerParams(use_tc_tiling_on_sc=False),
    )
    def seg(p_hbm, src_hbm, dst_hbm, z_hbm, out_hbm, src_v, dst_v, rows,
            sems, ssems, acc_sh, tab_sh):
        c = lax.axis_index("c")
        s = lax.axis_index("s")
        w = s * NC + c
        pltpu.sync_copy(z_hbm, acc_sh.at[pl.ds(s * ZR, ZR)])
        pltpu.sync_copy(p_hbm.at[pl.ds(s * (N // NS), N // NS)],
                        tab_sh.at[pl.ds(s * (N // NS), N // NS)])
        pltpu.sync_copy(src_hbm.at[pl.ds(w * CPT, CPT)], src_v)
        pltpu.sync_copy(dst_hbm.at[pl.ds(w * CPT, CPT)], dst_v)
        plsc.subcore_barrier()

        for b in range(NBUF - 1):
            pltpu.async_copy(tab_sh.at[src_v.at[b]], rows[b], sems[b])

        def body(i, carry):
            j0 = i * NBUF
            for b in range(NBUF):
                j = j0 + b
                bn = (b + NBUF - 1) % NBUF

                @pl.when(j + NBUF - 1 < CPT)
                def _():
                    @pl.when(j > 0)
                    def _():
                        pltpu.make_async_copy(
                            rows[bn], acc_sh.at[dst_v.at[j - 1]],
                            ssems[bn]).wait()
                    pltpu.async_copy(tab_sh.at[src_v.at[j + NBUF - 1]],
                                     rows[bn], sems[bn])

                pltpu.make_async_copy(tab_sh.at[src_v.at[j]], rows[b],
                                      sems[b]).wait()
                pltpu.async_copy(rows[b], acc_sh.at[dst_v.at[j]], ssems[b],
                                 add=True)
            return carry

        lax.fori_loop(0, CPT // NBUF, body, 0)
        for b in range(NBUF):
            j = CPT - NBUF + b
            pltpu.make_async_copy(rows[b % NBUF], acc_sh.at[dst_v.at[j]],
                                  ssems[j % NBUF]).wait()
        plsc.subcore_barrier()
        pltpu.sync_copy(acc_sh.at[pl.ds(s * ZR, ZR)],
                        out_hbm.at[c, pl.ds(s * ZR, ZR)])

    return seg


def kernel(x, edge_index, W1l, b1l, W1r, W2l, b2l, W2r):
    src = edge_index[0].astype(jnp.int32)
    dst = edge_index[1].astype(jnp.int32)
    pad = EPAD - E
    src = jnp.concatenate([src, jnp.zeros((pad,), jnp.int32)]).reshape(-1, CH)
    dst = jnp.concatenate([dst, jnp.full((pad,), N, jnp.int32)]).reshape(-1, CH)

    w1aug = jnp.concatenate([W1l, jnp.zeros((F, D1 - F), jnp.float32)], axis=1)
    e144 = jnp.concatenate(
        [jnp.zeros((1, F), jnp.float32), jnp.ones((1, 1), jnp.float32),
         jnp.zeros((1, D1 - F - 1), jnp.float32)], axis=1)
    z144 = jnp.zeros((ZR, D1), jnp.bfloat16)
    z128 = jnp.zeros((ZR, F), jnp.bfloat16)

    p1, r1 = _tc1(x, w1aug, e144, W1r)
    s1 = _make_segsum(D1)(p1, src, dst, z144)
    p2, r2, degb = _tc2(s1, r1, W2l, W2r, b1l.reshape(1, F))
    s2 = _make_segsum(F)(p2, src, dst, z128)
    return _tc3(s2, r2, degb, b2l.reshape(1, F))


# speedup vs baseline: 1.0173x; 1.0173x over previous
import functools

import jax
import jax.numpy as jnp
from jax import lax
from jax.experimental import pallas as pl
from jax.experimental.pallas import tpu as pltpu
from jax.experimental.pallas import tpu_sc as plsc

N = 10000
E = 320000
F = 128
D1 = 160
NC, NS = 2, 16
CH = 32
CPT = 320
NBUF = 4
EPAD = NC * NS * CPT * CH
ACC_ROWS = 10016
ZR = 626
RB = 2000

_P = jax.lax.Precision.HIGHEST


def _tc1_body(x_ref, wl_ref, e_ref, wr_ref, p_ref, r_ref):
    x = x_ref[...]
    p = lax.dot(x.astype(jnp.bfloat16), wl_ref[...].astype(jnp.bfloat16),
                preferred_element_type=jnp.float32) + e_ref[...]
    p_ref[...] = p.astype(jnp.bfloat16)
    r_ref[...] = lax.dot(x, wr_ref[...], precision=_P)


def _tc1(x, w1aug, e144, w1r):
    return pl.pallas_call(
        _tc1_body,
        grid=(N // RB,),
        in_specs=[
            pl.BlockSpec((RB, F), lambda i: (i, 0)),
            pl.BlockSpec((F, D1), lambda i: (0, 0)),
            pl.BlockSpec((1, D1), lambda i: (0, 0)),
            pl.BlockSpec((F, F), lambda i: (0, 0)),
        ],
        out_specs=[
            pl.BlockSpec((RB, D1), lambda i: (i, 0)),
            pl.BlockSpec((RB, F), lambda i: (i, 0)),
        ],
        out_shape=[
            jax.ShapeDtypeStruct((N, D1), jnp.bfloat16),
            jax.ShapeDtypeStruct((N, F), jnp.float32),
        ],
    )(x, w1aug, e144, w1r)


def _tc2_body(s_ref, r_ref, wl_ref, wr_ref, b1_ref, p_ref, r2_ref, deg_ref):
    s = s_ref[0].astype(jnp.float32) + s_ref[1].astype(jnp.float32)
    deg = s[:, F:F + 1]
    mean = s[:, :F] / jnp.maximum(deg, 1.0)
    h = jnp.maximum(mean + r_ref[...] + b1_ref[...], 0.0)
    p_ref[...] = lax.dot(h.astype(jnp.bfloat16),
                         wl_ref[...].astype(jnp.bfloat16),
                         preferred_element_type=jnp.float32
                         ).astype(jnp.bfloat16)
    r2_ref[...] = lax.dot(h, wr_ref[...], precision=_P)
    deg_ref[...] = jnp.broadcast_to(deg, (RB, F))


def _tc2(s1, r1, w2l, w2r, b1):
    return pl.pallas_call(
        _tc2_body,
        grid=(N // RB,),
        in_specs=[
            pl.BlockSpec((2, RB, D1), lambda i: (0, i, 0)),
            pl.BlockSpec((RB, F), lambda i: (i, 0)),
            pl.BlockSpec((F, F), lambda i: (0, 0)),
            pl.BlockSpec((F, F), lambda i: (0, 0)),
            pl.BlockSpec((1, F), lambda i: (0, 0)),
        ],
        out_specs=[
            pl.BlockSpec((RB, F), lambda i: (i, 0)),
            pl.BlockSpec((RB, F), lambda i: (i, 0)),
            pl.BlockSpec((RB, F), lambda i: (i, 0)),
        ],
        out_shape=[
            jax.ShapeDtypeStruct((N, F), jnp.bfloat16),
            jax.ShapeDtypeStruct((N, F), jnp.float32),
            jax.ShapeDtypeStruct((N, F), jnp.float32),
        ],
    )(s1, r1, w2l, w2r, b1)


def _tc3_body(s_ref, r_ref, deg_ref, b2_ref, o_ref):
    s = s_ref[0].astype(jnp.float32) + s_ref[1].astype(jnp.float32)
    o = s / jnp.maximum(deg_ref[...], 1.0) + r_ref[...] + b2_ref[...]
    m = jnp.max(o, axis=1, keepdims=True)
    z = o - m
    o_ref[...] = z - jnp.log(jnp.sum(jnp.exp(z), axis=1, keepdims=True))


def _tc3(s2, r2, degb, b2):
    return pl.pallas_call(
        _tc3_body,
        grid=(N // RB,),
        in_specs=[
            pl.BlockSpec((2, RB, F), lambda i: (0, i, 0)),
            pl.BlockSpec((RB, F), lambda i: (i, 0)),
            pl.BlockSpec((RB, F), lambda i: (i, 0)),
            pl.BlockSpec((1, F), lambda i: (0, 0)),
        ],
        out_specs=pl.BlockSpec((RB, F), lambda i: (i, 0)),
        out_shape=jax.ShapeDtypeStruct((N, F), jnp.float32),
    )(s2, r2, degb, b2)


@functools.cache
def _make_segsum(d):
    mesh = plsc.VectorSubcoreMesh(core_axis_name="c", subcore_axis_name="s",
                                  num_cores=NC, num_subcores=NS)

    @functools.partial(
        pl.kernel,
        out_type=jax.ShapeDtypeStruct((NC, ACC_ROWS, d), jnp.bfloat16),
        mesh=mesh,
        scratch_types=[
            pltpu.VMEM((CPT, CH), jnp.int32),
            pltpu.VMEM((CPT, CH), jnp.int32),
            [pltpu.VMEM((CH, d), jnp.bfloat16) for _ in range(NBUF)],
            [pltpu.SemaphoreType.DMA for _ in range(NBUF)],
            [pltpu.SemaphoreType.DMA for _ in range(NBUF)],
            pltpu.VMEM_SHARED((ACC_ROWS, d), jnp.bfloat16),
            pltpu.VMEM_SHARED((N, d), jnp.bfloat16),
        ],
        compiler_params=pltpu.CompilerParams(use_tc_tiling_on_sc=False),
    )
    def seg(p_hbm, src_hbm, dst_hbm, z_hbm, out_hbm, src_v, dst_v, rows,
            sems, ssems, acc_sh, tab_sh):
        c = lax.axis_index("c")
        s = lax.axis_index("s")
        w = s * NC + c
        pltpu.sync_copy(z_hbm, acc_sh.at[pl.ds(s * ZR, ZR)])
        pltpu.sync_copy(p_hbm.at[pl.ds(s * (N // NS), N // NS)],
                        tab_sh.at[pl.ds(s * (N // NS), N // NS)])
        pltpu.sync_copy(src_hbm.at[pl.ds(w * CPT, CPT)], src_v)
        pltpu.sync_copy(dst_hbm.at[pl.ds(w * CPT, CPT)], dst_v)
        plsc.subcore_barrier()

        for b in range(NBUF - 1):
            pltpu.async_copy(tab_sh.at[src_v.at[b]], rows[b], sems[b])

        def body(i, carry):
            j0 = i * NBUF
            for b in range(NBUF):
                j = j0 + b
                bn = (b + NBUF - 1) % NBUF

                @pl.when(j + NBUF - 1 < CPT)
                def _():
                    @pl.when(j > 0)
                    def _():
                        pltpu.make_async_copy(
                            rows[bn], acc_sh.at[dst_v.at[j - 1]],
                            ssems[bn]).wait()
                    pltpu.async_copy(tab_sh.at[src_v.at[j + NBUF - 1]],
                                     rows[bn], sems[bn])

                pltpu.make_async_copy(tab_sh.at[src_v.at[j]], rows[b],
                                      sems[b]).wait()
                pltpu.async_copy(rows[b], acc_sh.at[dst_v.at[j]], ssems[b],
                                 add=True)
            return carry

        lax.fori_loop(0, CPT // NBUF, body, 0)
        for b in range(NBUF):
            j = CPT - NBUF + b
            pltpu.make_async_copy(rows[b % NBUF], acc_sh.at[dst_v.at[j]],
                                  ssems[j % NBUF]).wait()
        plsc.subcore_barrier()
        pltpu.sync_copy(acc_sh.at[pl.ds(s * ZR, ZR)],
                        out_hbm.at[c, pl.ds(s * ZR, ZR)])

    return seg


def kernel(x, edge_index, W1l, b1l, W1r, W2l, b2l, W2r):
    src = edge_index[0].astype(jnp.int32)
    dst = edge_index[1].astype(jnp.int32)
    pad = EPAD - E
    src = jnp.concatenate([src, jnp.zeros((pad,), jnp.int32)]).reshape(-1, CH)
    dst = jnp.concatenate([dst, jnp.full((pad,), N, jnp.int32)]).reshape(-1, CH)

    w1aug = jnp.concatenate([W1l, jnp.zeros((F, D1 - F), jnp.float32)], axis=1)
    e144 = jnp.concatenate(
        [jnp.zeros((1, F), jnp.float32), jnp.ones((1, 1), jnp.float32),
         jnp.zeros((1, D1 - F - 1), jnp.float32)], axis=1)
    z144 = jnp.zeros((ZR, D1), jnp.bfloat16)
    z128 = jnp.zeros((ZR, F), jnp.bfloat16)

    p1, r1 = _tc1(x, w1aug, e144, W1r)
    s1 = _make_segsum(D1)(p1, src, dst, z144)
    p2, r2, degb = _tc2(s1, r1, W2l, W2r, b1l.reshape(1, F))
    s2 = _make_segsum(F)(p2, src, dst, z128)
    return _tc3(s2, r2, degb, b2l.reshape(1, F))
